# Initial kernel scaffold; baseline (speedup 1.0000x reference)
import jax
import jax.numpy as jnp
from jax.experimental import pallas as pl


def kernel(o_wh, t_mask, t_ind, t_wh):
    raise NotImplementedError("write your pallas kernel here")


# speedup vs baseline: 1.2690x; 1.2690x over previous
import jax
import jax.numpy as jnp
from jax import lax
from jax.experimental import pallas as pl
from jax.experimental.pallas import tpu as pltpu
from jax.experimental.pallas import tpu_sc as plsc

_NC, _NS, _L = 2, 16, 16
_KP = 512


def _make_sc_loss(B, C, HW):
    assert B == _NC * _NS
    n_idx_rows = C * _KP // 128
    n_chunks = _KP // _L
    mesh = plsc.VectorSubcoreMesh(core_axis_name="c", subcore_axis_name="s")

    def body(o_flat, ind_p, twh_p, mask_p, out, idx_v, idxc_v, vals_v,
             twh_v, mask_v, outb_v, sem):
        b = lax.axis_index("s") * _NC + lax.axis_index("c")
        base0 = b * (C * HW)

        pltpu.sync_copy(ind_p.at[b], idx_v)

        for j in range(n_chunks):
            iv = idx_v[pl.ds(_L * j, _L)]
            r, o = j // 8, _L * (j % 8)
            idxc_v[r, pl.ds(o, _L)] = iv + base0
            idxc_v[n_idx_rows // 2 + r, pl.ds(o, _L)] = iv + (base0 + HW)

        copies = [
            pltpu.async_copy(o_flat.at[idxc_v.at[i]], vals_v.at[i], sem)
            for i in range(n_idx_rows)
        ]
        pltpu.sync_copy(twh_p.at[b], twh_v)
        pltpu.sync_copy(mask_p.at[b], mask_v)
        for cp in copies:
            cp.wait()

        acc = jnp.zeros((_L,), jnp.float32)
        smv = jnp.zeros((_L,), jnp.float32)
        for j in range(n_chunks):
            r, o = j // 8, _L * (j % 8)
            m = mask_v[pl.ds(_L * j, _L)]
            v0 = vals_v[r, pl.ds(o, _L)]
            v1 = vals_v[n_idx_rows // 2 + r, pl.ds(o, _L)]
            t0 = twh_v[pl.ds(_L * j, _L)]
            t1 = twh_v[pl.ds(_KP + _L * j, _L)]
            acc = acc + jnp.abs((v0 - t0) * m) + jnp.abs((v1 - t1) * m)
            smv = smv + m

        outb_v[0, pl.ds(0, _L)] = acc
        outb_v[1, pl.ds(0, _L)] = smv
        pltpu.sync_copy(outb_v, out.at[b])

    return pl.kernel(
        body,
        mesh=mesh,
        out_type=jax.ShapeDtypeStruct((B, 2, _L), jnp.float32),
        scratch_types=[
            pltpu.VMEM((_KP,), jnp.int32),
            pltpu.VMEM((n_idx_rows, 128), jnp.int32),
            pltpu.VMEM((n_idx_rows, 128), jnp.float32),
            pltpu.VMEM((C * _KP,), jnp.float32),
            pltpu.VMEM((_KP,), jnp.float32),
            pltpu.VMEM((2, _L), jnp.float32),
            pltpu.SemaphoreType.DMA,
        ],
    )


def kernel(o_wh, t_mask, t_ind, t_wh):
    B, C, H, W = o_wh.shape
    K = t_ind.shape[1]
    o_flat = o_wh.reshape(-1)
    ind_p = jnp.pad(t_ind.astype(jnp.int32), ((0, 0), (0, _KP - K)))
    mask_p = jnp.pad(t_mask, ((0, 0), (0, _KP - K)))
    twh_p = jnp.pad(
        jnp.transpose(t_wh, (0, 2, 1)), ((0, 0), (0, 0), (0, _KP - K))
    ).reshape(B, C * _KP)
    out = _make_sc_loss(B, C, H * W)(o_flat, ind_p, twh_p, mask_p)
    return out[:, 0, :].sum() / out[:, 1, :].sum()


# speedup vs baseline: 1.5448x; 1.2174x over previous
import jax
import jax.numpy as jnp
from jax import lax
from jax.experimental import pallas as pl
from jax.experimental.pallas import tpu as pltpu
from jax.experimental.pallas import tpu_sc as plsc

_NC, _NS, _L = 2, 16, 16
_KP = 512
_HCHUNK = 64


def _make_sc_loss(B, C, H, W):
    assert B == _NC * _NS and C == 2 and W & (W - 1) == 0
    w_shift = (W - 1).bit_length()
    n_chunks = H // _HCHUNK
    n_kchunks = _KP // _L
    rows_per_b = C * H
    mesh = plsc.VectorSubcoreMesh(core_axis_name="c", subcore_axis_name="s")

    def body(o2d, ind_p, twh_p, mask_p, out, idx_v, twh_v, mask_v,
             buf0, buf1, outb_v, sem0, sem1):
        b = lax.axis_index("s") * _NC + lax.axis_index("c")
        rbase = b * rows_per_b

        pltpu.sync_copy(ind_p.at[b], idx_v)
        pltpu.sync_copy(twh_p.at[b], twh_v)
        pltpu.sync_copy(mask_p.at[b], mask_v)

        bufs, sems = (buf0, buf1), (sem0, sem1)

        def issue(g):
            bf, sm = bufs[g % 2], sems[g % 2]
            h0 = _HCHUNK * g
            c0 = pltpu.async_copy(
                o2d.at[pl.ds(rbase + h0, _HCHUNK)],
                bf.at[pl.ds(0, _HCHUNK)], sm)
            c1 = pltpu.async_copy(
                o2d.at[pl.ds(rbase + H + h0, _HCHUNK)],
                bf.at[pl.ds(_HCHUNK, _HCHUNK)], sm)
            return c0, c1

        pend = issue(0)
        acc = jnp.zeros((_L,), jnp.float32)
        smv = jnp.zeros((_L,), jnp.float32)
        for g in range(n_chunks):
            nxt = issue(g + 1) if g + 1 < n_chunks else None
            for cp in pend:
                cp.wait()
            bf = bufs[g % 2]
            for j in range(n_kchunks):
                p = idx_v[pl.ds(_L * j, _L)]
                m = mask_v[pl.ds(_L * j, _L)]
                h = p >> w_shift
                w = p & (W - 1)
                rloc = h - _HCHUNK * g
                inr = (rloc >= 0) & (rloc < _HCHUNK)
                rc = jnp.minimum(jnp.maximum(rloc, 0), _HCHUNK - 1)
                v0 = plsc.load_gather(bf, [rc, w])
                v1 = plsc.load_gather(bf, [rc + _HCHUNK, w])
                t0 = twh_v[pl.ds(_L * j, _L)]
                t1 = twh_v[pl.ds(_KP + _L * j, _L)]
                mm = jnp.where(inr, m, jnp.float32(0.0))
                acc = acc + jnp.abs((v0 - t0) * mm) + jnp.abs((v1 - t1) * mm)
                if g == 0:
                    smv = smv + m
            pend = nxt

        outb_v[0, pl.ds(0, _L)] = acc
        outb_v[1, pl.ds(0, _L)] = smv
        pltpu.sync_copy(outb_v, out.at[b])

    return pl.kernel(
        body,
        mesh=mesh,
        out_type=jax.ShapeDtypeStruct((B, 2, _L), jnp.float32),
        scratch_types=[
            pltpu.VMEM((_KP,), jnp.int32),
            pltpu.VMEM((C * _KP,), jnp.float32),
            pltpu.VMEM((_KP,), jnp.float32),
            pltpu.VMEM((C * _HCHUNK, W), jnp.float32),
            pltpu.VMEM((C * _HCHUNK, W), jnp.float32),
            pltpu.VMEM((2, _L), jnp.float32),
            pltpu.SemaphoreType.DMA,
            pltpu.SemaphoreType.DMA,
        ],
        compiler_params=pltpu.CompilerParams(needs_layout_passes=False),
    )


def kernel(o_wh, t_mask, t_ind, t_wh):
    B, C, H, W = o_wh.shape
    K = t_ind.shape[1]
    o2d = o_wh.reshape(B * C * H, W)
    ind_p = jnp.pad(t_ind.astype(jnp.int32), ((0, 0), (0, _KP - K)))
    mask_p = jnp.pad(t_mask, ((0, 0), (0, _KP - K)))
    twh_p = jnp.pad(
        jnp.transpose(t_wh, (0, 2, 1)), ((0, 0), (0, 0), (0, _KP - K))
    ).reshape(B, C * _KP)
    out = _make_sc_loss(B, C, H, W)(o2d, ind_p, twh_p, mask_p)
    return out[:, 0, :].sum() / out[:, 1, :].sum()
